# baseline (speedup 1.0000x reference)
import functools

import jax
import jax.numpy as jnp
from jax import lax
from jax.experimental import pallas as pl
from jax.experimental.pallas import tpu as pltpu
from jax.experimental.pallas import tpu_sc as plsc

L = 16
NC = 2
NS = 16
NW = NC * NS

H = 2048
W = 2048
P = 4 * 512 * 512
PPW = P // NW
CH = 2048
NCHUNK = PPW // CH
ROWS = CH // 128
VECS = CH // L
VPR = 128 // L


def _tex_body(x_hbm, tex_hbm, out_hbm,
              xbuf, i00, i01, i10, i11,
              v00, v01, v10, v11,
              wxb, wyb, obuf, sem):
    wid = lax.axis_index("s") * NC + lax.axis_index("c")
    base = wid * PPW

    def chunk_body(c, carry):
        off = base + c * CH
        pltpu.sync_copy(x_hbm.at[pl.ds(off * 2, CH * 2)], xbuf)

        lane = lax.iota(jnp.int32, L)

        def vec_body(i, carry2):
            gx_idx = i * (2 * L) + 2 * lane
            gx = plsc.load_gather(xbuf, [gx_idx])
            gy = plsc.load_gather(xbuf, [gx_idx + 1])
            ix = ((gx + 1.0) * jnp.float32(W) - 1.0) * 0.5
            iy = ((gy + 1.0) * jnp.float32(H) - 1.0) * 0.5
            ix = jnp.minimum(jnp.maximum(ix, 0.0), jnp.float32(W - 1))
            iy = jnp.minimum(jnp.maximum(iy, 0.0), jnp.float32(H - 1))
            ix0 = ix.astype(jnp.int32)
            iy0 = iy.astype(jnp.int32)
            wx1 = ix - ix0.astype(jnp.float32)
            wy1 = iy - iy0.astype(jnp.float32)
            ix1 = jnp.minimum(ix0 + 1, W - 1)
            iy1 = jnp.minimum(iy0 + 1, H - 1)
            row0 = iy0 << 11
            row1 = iy1 << 11
            r = i // VPR
            col = (i % VPR) * L
            csl = pl.ds(col, L)
            i00[r, csl] = row0 + ix0
            i01[r, csl] = row0 + ix1
            i10[r, csl] = row1 + ix0
            i11[r, csl] = row1 + ix1
            b = pl.ds(i * L, L)
            wxb[b] = wx1
            wyb[b] = wy1
            return carry2

        lax.fori_loop(0, VECS, vec_body, 0, unroll=2)

        cps = []
        for idx_ref, v_ref in ((i00, v00), (i01, v01), (i10, v10), (i11, v11)):
            for j in range(ROWS):
                cps.append(pltpu.async_copy(tex_hbm.at[idx_ref.at[j]],
                                            v_ref.at[j], sem))
        for cp in cps:
            cp.wait()

        def comb_body(i, carry2):
            r = i // VPR
            csl = pl.ds((i % VPR) * L, L)
            b = pl.ds(i * L, L)
            wx1 = wxb[b]
            wy1 = wyb[b]
            a00 = v00[r, csl]
            a01 = v01[r, csl]
            a10 = v10[r, csl]
            a11 = v11[r, csl]
            top = a00 + (a01 - a00) * wx1
            bot = a10 + (a11 - a10) * wx1
            obuf[b] = top + (bot - top) * wy1
            return carry2

        lax.fori_loop(0, VECS, comb_body, 0, unroll=2)

        pltpu.sync_copy(obuf, out_hbm.at[pl.ds(off, CH)])
        return carry

    lax.fori_loop(0, NCHUNK, chunk_body, 0)


@functools.partial(jax.jit, static_argnames=())
def kernel(x, layer1):
    batch = x.shape[0]
    x_flat = x.reshape(-1)
    tex_flat = layer1.reshape(-1)

    run = pl.kernel(
        _tex_body,
        out_type=jax.ShapeDtypeStruct((P,), jnp.float32),
        mesh=plsc.VectorSubcoreMesh(core_axis_name="c", subcore_axis_name="s"),
        scratch_types=[
            pltpu.VMEM((CH * 2,), jnp.float32),
            pltpu.VMEM((ROWS, 128), jnp.int32),
            pltpu.VMEM((ROWS, 128), jnp.int32),
            pltpu.VMEM((ROWS, 128), jnp.int32),
            pltpu.VMEM((ROWS, 128), jnp.int32),
            pltpu.VMEM((ROWS, 128), jnp.float32),
            pltpu.VMEM((ROWS, 128), jnp.float32),
            pltpu.VMEM((ROWS, 128), jnp.float32),
            pltpu.VMEM((ROWS, 128), jnp.float32),
            pltpu.VMEM((CH,), jnp.float32),
            pltpu.VMEM((CH,), jnp.float32),
            pltpu.VMEM((CH,), jnp.float32),
            pltpu.SemaphoreType.DMA,
        ],
        compiler_params=pltpu.CompilerParams(needs_layout_passes=False),
    )
    y = run(x_flat, tex_flat)
    return y.reshape(batch, 1, x.shape[1], x.shape[2])


# speedup vs baseline: 1.0003x; 1.0003x over previous
import functools

import jax
import jax.numpy as jnp
from jax import lax
from jax.experimental import pallas as pl
from jax.experimental.pallas import tpu as pltpu
from jax.experimental.pallas import tpu_sc as plsc

L = 16
NC = 2
NS = 16
NW = NC * NS

H = 2048
W = 2048
P = 4 * 512 * 512
PPW = P // NW
CH = 2048
NCHUNK = PPW // CH
ROWS = CH // 128
VECS = CH // L
VPR = 128 // L


def _untile_body(t_ref, out_ref):
    out_ref[...] = t_ref[...].reshape(128, 128)


def _untile(tex2d):
    out = pl.pallas_call(
        _untile_body,
        grid=(H // 8,),
        in_specs=[pl.BlockSpec((8, W), lambda i: (i, 0))],
        out_specs=pl.BlockSpec((128, 128), lambda i: (i, 0)),
        out_shape=jax.ShapeDtypeStruct((H * W // 128, 128), jnp.float32),
    )(tex2d)
    return out.reshape(H * W)


def _tex_body(x_hbm, tex_hbm, out_hbm,
              xbuf, i00, i01, i10, i11,
              v00, v01, v10, v11,
              wxb, wyb, obuf, sem):
    wid = lax.axis_index("s") * NC + lax.axis_index("c")
    base = wid * PPW

    def chunk_body(c, carry):
        off = base + c * CH
        pltpu.sync_copy(x_hbm.at[pl.ds(off * 2, CH * 2)], xbuf)

        lane = lax.iota(jnp.int32, L)

        def vec_body(i, carry2):
            gx_idx = i * (2 * L) + 2 * lane
            gx = plsc.load_gather(xbuf, [gx_idx])
            gy = plsc.load_gather(xbuf, [gx_idx + 1])
            ix = ((gx + 1.0) * jnp.float32(W) - 1.0) * 0.5
            iy = ((gy + 1.0) * jnp.float32(H) - 1.0) * 0.5
            ix = jnp.minimum(jnp.maximum(ix, 0.0), jnp.float32(W - 1))
            iy = jnp.minimum(jnp.maximum(iy, 0.0), jnp.float32(H - 1))
            ix0 = ix.astype(jnp.int32)
            iy0 = iy.astype(jnp.int32)
            wx1 = ix - ix0.astype(jnp.float32)
            wy1 = iy - iy0.astype(jnp.float32)
            ix1 = jnp.minimum(ix0 + 1, W - 1)
            iy1 = jnp.minimum(iy0 + 1, H - 1)
            row0 = iy0 << 11
            row1 = iy1 << 11
            r = i // VPR
            csl = pl.ds((i % VPR) * L, L)
            i00[r, csl] = row0 + ix0
            i01[r, csl] = row0 + ix1
            i10[r, csl] = row1 + ix0
            i11[r, csl] = row1 + ix1
            b = pl.ds(i * L, L)
            wxb[b] = wx1
            wyb[b] = wy1
            return carry2

        lax.fori_loop(0, VECS, vec_body, 0, unroll=2)

        cps = []
        for idx_ref, v_ref in ((i00, v00), (i01, v01), (i10, v10), (i11, v11)):
            for j in range(ROWS):
                cps.append(pltpu.async_copy(tex_hbm.at[idx_ref.at[j]],
                                            v_ref.at[j], sem))
        for cp in cps:
            cp.wait()

        def comb_body(i, carry2):
            r = i // VPR
            csl = pl.ds((i % VPR) * L, L)
            b = pl.ds(i * L, L)
            wx1 = wxb[b]
            wy1 = wyb[b]
            a00 = v00[r, csl]
            a01 = v01[r, csl]
            a10 = v10[r, csl]
            a11 = v11[r, csl]
            top = a00 + (a01 - a00) * wx1
            bot = a10 + (a11 - a10) * wx1
            obuf[b] = top + (bot - top) * wy1
            return carry2

        lax.fori_loop(0, VECS, comb_body, 0, unroll=2)

        pltpu.sync_copy(obuf, out_hbm.at[pl.ds(off, CH)])
        return carry

    lax.fori_loop(0, NCHUNK, chunk_body, 0)


def kernel(x, layer1):
    batch = x.shape[0]
    x_flat = x.reshape(-1)
    tex_flat = _untile(layer1.reshape(H, W))

    run = pl.kernel(
        _tex_body,
        out_type=jax.ShapeDtypeStruct((P,), jnp.float32),
        mesh=plsc.VectorSubcoreMesh(core_axis_name="c", subcore_axis_name="s"),
        scratch_types=[
            pltpu.VMEM((CH * 2,), jnp.float32),
            pltpu.VMEM((ROWS, 128), jnp.int32),
            pltpu.VMEM((ROWS, 128), jnp.int32),
            pltpu.VMEM((ROWS, 128), jnp.int32),
            pltpu.VMEM((ROWS, 128), jnp.int32),
            pltpu.VMEM((ROWS, 128), jnp.float32),
            pltpu.VMEM((ROWS, 128), jnp.float32),
            pltpu.VMEM((ROWS, 128), jnp.float32),
            pltpu.VMEM((ROWS, 128), jnp.float32),
            pltpu.VMEM((CH,), jnp.float32),
            pltpu.VMEM((CH,), jnp.float32),
            pltpu.VMEM((CH,), jnp.float32),
            pltpu.SemaphoreType.DMA,
        ],
        compiler_params=pltpu.CompilerParams(needs_layout_passes=False),
    )
    y = run(x_flat, tex_flat)
    return y.reshape(batch, 1, x.shape[1], x.shape[2])


# speedup vs baseline: 2.4936x; 2.4928x over previous
import jax
import jax.numpy as jnp
from jax import lax
from jax.experimental import pallas as pl
from jax.experimental.pallas import tpu as pltpu
from jax.experimental.pallas import tpu_sc as plsc

L = 16
NC = 2
NS = 16
NW = NC * NS

H = 2048
W = 2048
P = 4 * 512 * 512
PPW = P // NW
CH = 2048
NCHUNK = PPW // CH
ROWS = CH // 128
VECS = CH // L
VPR = 128 // L


def _untile_body(t_ref, out_ref):
    r, c = t_ref.shape
    out_ref[...] = t_ref[...].reshape(r * c // 128, 128)


def _untile(a2d):
    rows, cols = a2d.shape
    out = pl.pallas_call(
        _untile_body,
        grid=(rows // 8,),
        in_specs=[pl.BlockSpec((8, cols), lambda i: (i, 0))],
        out_specs=pl.BlockSpec((8 * cols // 128, 128), lambda i: (i, 0)),
        out_shape=jax.ShapeDtypeStruct((rows * cols // 128, 128), jnp.float32),
    )(a2d)
    return out.reshape(rows * cols)


def _tex_body(gx_hbm, gy_hbm, tex_hbm, out_hbm,
              gxbuf, gybuf, i00, i01, i10, i11,
              v00, v01, v10, v11,
              wxb, wyb, obuf, sem):
    wid = lax.axis_index("s") * NC + lax.axis_index("c")
    base = wid * PPW

    def chunk_body(c, carry):
        off = base + c * CH
        pltpu.sync_copy(gx_hbm.at[pl.ds(off, CH)], gxbuf)
        pltpu.sync_copy(gy_hbm.at[pl.ds(off, CH)], gybuf)

        def vec_body(i, carry2):
            b = pl.ds(i * L, L)
            gx = gxbuf[b]
            gy = gybuf[b]
            ix = ((gx + 1.0) * jnp.float32(W) - 1.0) * 0.5
            iy = ((gy + 1.0) * jnp.float32(H) - 1.0) * 0.5
            ix = jnp.minimum(jnp.maximum(ix, 0.0), jnp.float32(W - 1))
            iy = jnp.minimum(jnp.maximum(iy, 0.0), jnp.float32(H - 1))
            ix0 = ix.astype(jnp.int32)
            iy0 = iy.astype(jnp.int32)
            wx1 = ix - ix0.astype(jnp.float32)
            wy1 = iy - iy0.astype(jnp.float32)
            ix1 = jnp.minimum(ix0 + 1, W - 1)
            iy1 = jnp.minimum(iy0 + 1, H - 1)
            row0 = iy0 << 11
            row1 = iy1 << 11
            r = i // VPR
            csl = pl.ds((i % VPR) * L, L)
            i00[r, csl] = row0 + ix0
            i01[r, csl] = row0 + ix1
            i10[r, csl] = row1 + ix0
            i11[r, csl] = row1 + ix1
            wxb[b] = wx1
            wyb[b] = wy1
            return carry2

        lax.fori_loop(0, VECS, vec_body, 0, unroll=2)

        cps = []
        for idx_ref, v_ref in ((i00, v00), (i01, v01), (i10, v10), (i11, v11)):
            for j in range(ROWS):
                cps.append(pltpu.async_copy(tex_hbm.at[idx_ref.at[j]],
                                            v_ref.at[j], sem))
        for cp in cps:
            cp.wait()

        def comb_body(i, carry2):
            r = i // VPR
            csl = pl.ds((i % VPR) * L, L)
            b = pl.ds(i * L, L)
            wx1 = wxb[b]
            wy1 = wyb[b]
            a00 = v00[r, csl]
            a01 = v01[r, csl]
            a10 = v10[r, csl]
            a11 = v11[r, csl]
            top = a00 + (a01 - a00) * wx1
            bot = a10 + (a11 - a10) * wx1
            obuf[b] = top + (bot - top) * wy1
            return carry2

        lax.fori_loop(0, VECS, comb_body, 0, unroll=2)

        pltpu.sync_copy(obuf, out_hbm.at[pl.ds(off, CH)])
        return carry

    lax.fori_loop(0, NCHUNK, chunk_body, 0)


def kernel(x, layer1):
    batch = x.shape[0]
    gx_flat = _untile(x[..., 0].reshape(batch * x.shape[1], x.shape[2]))
    gy_flat = _untile(x[..., 1].reshape(batch * x.shape[1], x.shape[2]))
    tex_flat = _untile(layer1.reshape(H, W))

    run = pl.kernel(
        _tex_body,
        out_type=jax.ShapeDtypeStruct((P,), jnp.float32),
        mesh=plsc.VectorSubcoreMesh(core_axis_name="c", subcore_axis_name="s"),
        scratch_types=[
            pltpu.VMEM((CH,), jnp.float32),
            pltpu.VMEM((CH,), jnp.float32),
            pltpu.VMEM((ROWS, 128), jnp.int32),
            pltpu.VMEM((ROWS, 128), jnp.int32),
            pltpu.VMEM((ROWS, 128), jnp.int32),
            pltpu.VMEM((ROWS, 128), jnp.int32),
            pltpu.VMEM((ROWS, 128), jnp.float32),
            pltpu.VMEM((ROWS, 128), jnp.float32),
            pltpu.VMEM((ROWS, 128), jnp.float32),
            pltpu.VMEM((ROWS, 128), jnp.float32),
            pltpu.VMEM((CH,), jnp.float32),
            pltpu.VMEM((CH,), jnp.float32),
            pltpu.VMEM((CH,), jnp.float32),
            pltpu.SemaphoreType.DMA,
        ],
        compiler_params=pltpu.CompilerParams(needs_layout_passes=False),
    )
    y = run(gx_flat, gy_flat, tex_flat)
    return y.reshape(batch, 1, x.shape[1], x.shape[2])


# speedup vs baseline: 7.2482x; 2.9067x over previous
import jax
import jax.numpy as jnp
from jax import lax
from jax.experimental import pallas as pl
from jax.experimental.pallas import tpu as pltpu
from jax.experimental.pallas import tpu_sc as plsc

L = 16
NC = 2
NS = 16
NW = NC * NS

H = 2048
W = 2048
P = 4 * 512 * 512
PPW = P // NW
CH = 2048
NCHUNK = PPW // CH
VECS = CH // L


def _tex_body(gx_hbm, gy_hbm, tex_hbm, out_hbm,
              gxA, gyA, i00A, i01A, i10A, i11A,
              v00A, v01A, v10A, v11A, wxA, wyA, obA,
              gxB, gyB, i00B, i01B, i10B, i11B,
              v00B, v01B, v10B, v11B, wxB, wyB, obB,
              semA, semB, semOutA, semOutB):
    wid = lax.axis_index("s") * NC + lax.axis_index("c")
    base = wid * PPW

    A = (gxA, gyA, (i00A, i01A, i10A, i11A), (v00A, v01A, v10A, v11A),
         wxA, wyA, obA, semA, semOutA)
    B = (gxB, gyB, (i00B, i01B, i10B, i11B), (v00B, v01B, v10B, v11B),
         wxB, wyB, obB, semB, semOutB)

    lane = lax.iota(jnp.int32, L)

    def load_compute(S, c):
        gxb, gyb, idx, _, wxb, wyb, _, _, _ = S
        off = base + c * CH
        pltpu.sync_copy(gx_hbm.at[pl.ds(off, CH)], gxb)
        pltpu.sync_copy(gy_hbm.at[pl.ds(off, CH)], gyb)

        def vec_body(i, carry):
            b = pl.ds(i * L, L)
            gx = gxb[b]
            gy = gyb[b]
            ix = ((gx + 1.0) * jnp.float32(W) - 1.0) * 0.5
            iy = ((gy + 1.0) * jnp.float32(H) - 1.0) * 0.5
            ix = jnp.minimum(jnp.maximum(ix, 0.0), jnp.float32(W - 1))
            iy = jnp.minimum(jnp.maximum(iy, 0.0), jnp.float32(H - 1))
            ix0 = ix.astype(jnp.int32)
            iy0 = iy.astype(jnp.int32)
            wxb[b] = ix - ix0.astype(jnp.float32)
            wyb[b] = iy - iy0.astype(jnp.float32)
            ix1 = jnp.minimum(ix0 + 1, W - 1)
            iy1 = jnp.minimum(iy0 + 1, H - 1)
            row0 = iy0 << 11
            row1 = iy1 << 11
            idx[0][b] = row0 + ix0
            idx[1][b] = row0 + ix1
            idx[2][b] = row1 + ix0
            idx[3][b] = row1 + ix1
            return carry

        lax.fori_loop(0, VECS, vec_body, 0, unroll=2)

    def fire(S):
        _, _, idx, v, _, _, _, sem, _ = S
        for k in range(4):
            pltpu.async_copy(tex_hbm.at[idx[k]], v[k], sem)

    def drain(S):
        _, _, idx, v, _, _, _, sem, _ = S
        for k in range(4):
            pltpu.make_async_copy(tex_hbm.at[idx[k]], v[k], sem).wait()

    def combine_out(S, c, first):
        _, _, _, v, wxb, wyb, obf, _, semo = S
        off = base + c * CH

        @pl.when(jnp.logical_not(first))
        def _():
            pltpu.make_async_copy(obf, out_hbm.at[pl.ds(off, CH)], semo).wait()

        def comb_body(i, carry):
            b = pl.ds(i * L, L)
            wx1 = wxb[b]
            wy1 = wyb[b]
            a00 = v[0][b]
            a01 = v[1][b]
            a10 = v[2][b]
            a11 = v[3][b]
            top = a00 + (a01 - a00) * wx1
            bot = a10 + (a11 - a10) * wx1
            obf[b] = top + (bot - top) * wy1
            return carry

        lax.fori_loop(0, VECS, comb_body, 0, unroll=2)
        pltpu.async_copy(obf, out_hbm.at[pl.ds(off, CH)], semo)

    load_compute(A, 0)
    fire(A)

    def pair_body(k, carry):
        cA = 2 * k
        load_compute(B, cA + 1)
        fire(B)
        drain(A)
        combine_out(A, cA, k == 0)

        @pl.when(k < NCHUNK // 2 - 1)
        def _():
            load_compute(A, cA + 2)
            fire(A)

        drain(B)
        combine_out(B, cA + 1, k == 0)
        return carry

    lax.fori_loop(0, NCHUNK // 2, pair_body, 0)

    last = base + (NCHUNK - 1) * CH
    pltpu.make_async_copy(obA, out_hbm.at[pl.ds(last, CH)], semOutA).wait()
    pltpu.make_async_copy(obB, out_hbm.at[pl.ds(last, CH)], semOutB).wait()


def kernel(x, layer1):
    batch = x.shape[0]
    gx_flat = x[..., 0].reshape(-1)
    gy_flat = x[..., 1].reshape(-1)
    tex_flat = layer1.reshape(-1)

    buf = lambda dt: pltpu.VMEM((CH,), dt)
    one_set = [buf(jnp.float32), buf(jnp.float32),
               buf(jnp.int32), buf(jnp.int32), buf(jnp.int32), buf(jnp.int32),
               buf(jnp.float32), buf(jnp.float32), buf(jnp.float32),
               buf(jnp.float32), buf(jnp.float32), buf(jnp.float32),
               buf(jnp.float32)]

    run = pl.kernel(
        _tex_body,
        out_type=jax.ShapeDtypeStruct((P,), jnp.float32),
        mesh=plsc.VectorSubcoreMesh(core_axis_name="c", subcore_axis_name="s"),
        scratch_types=one_set + one_set + [
            pltpu.SemaphoreType.DMA,
            pltpu.SemaphoreType.DMA,
            pltpu.SemaphoreType.DMA,
            pltpu.SemaphoreType.DMA,
        ],
        compiler_params=pltpu.CompilerParams(needs_layout_passes=False),
    )
    y = run(gx_flat, gy_flat, tex_flat)
    return y.reshape(batch, 1, x.shape[1], x.shape[2])
